# baseline (speedup 1.0000x reference)
import jax
import jax.numpy as jnp
from jax import lax
from jax.experimental import pallas as pl
from jax.experimental.pallas import tpu as pltpu
from jax.experimental.pallas import tpu_sc as plsc

D = 64
B, T = 4096, 200
NC, NS = 2, 16
NW = NC * NS
B_PER_W = B // NW
NCHUNK = B_PER_W
NBUF = 4
SPLIT = 128
SCALE = 8.0


def _body(table_hbm, idx_hbm, out_hbm, idx_bufs, row_bufs, gsems, ssems):
    wid = lax.axis_index("s") * NC + lax.axis_index("c")
    b0 = wid * B_PER_W

    def fire_gathers(g, b):
        pltpu.sync_copy(idx_hbm.at[pl.ds((b0 + g) * T, T)], idx_bufs[b])
        pltpu.async_copy(
            table_hbm.at[idx_bufs[b].at[pl.ds(0, SPLIT)]],
            row_bufs[b].at[pl.ds(0, SPLIT)],
            gsems[b],
        )
        pltpu.async_copy(
            table_hbm.at[idx_bufs[b].at[pl.ds(SPLIT, T - SPLIT)]],
            row_bufs[b].at[pl.ds(SPLIT, T - SPLIT)],
            gsems[b],
        )

    def wait_gathers(b):
        pltpu.make_async_copy(
            table_hbm.at[idx_bufs[b].at[pl.ds(0, SPLIT)]],
            row_bufs[b].at[pl.ds(0, SPLIT)],
            gsems[b],
        ).wait()
        pltpu.make_async_copy(
            table_hbm.at[idx_bufs[b].at[pl.ds(SPLIT, T - SPLIT)]],
            row_bufs[b].at[pl.ds(SPLIT, T - SPLIT)],
            gsems[b],
        ).wait()

    def wait_store(b):
        pltpu.make_async_copy(row_bufs[b], out_hbm.at[0], ssems[b]).wait()

    for b in range(NBUF - 1):
        fire_gathers(b, b)

    def chunk_iter(s, carry):
        for b in range(NBUF):
            g = s * NBUF + b
            wait_gathers(b)

            @plsc.parallel_loop(0, T, unroll=4)
            def scale_row(r):
                for dcol in range(D // 16):
                    sl = (r, pl.ds(dcol * 16, 16))
                    row_bufs[b][sl] = row_bufs[b][sl] * SCALE

            pltpu.async_copy(row_bufs[b], out_hbm.at[b0 + g], ssems[b])

            bb = (b + NBUF - 1) % NBUF

            @pl.when(g + NBUF - 1 < NCHUNK)
            def _prime():
                @pl.when(g >= 1)
                def _drain_store():
                    wait_store(bb)

                fire_gathers(g + NBUF - 1, bb)

        return carry

    lax.fori_loop(0, NCHUNK // NBUF, chunk_iter, 0)

    for b in range(NBUF):
        wait_store(b)


@jax.jit
def _emb(table, idx1d):
    mesh = plsc.VectorSubcoreMesh(core_axis_name="c", subcore_axis_name="s")
    return pl.kernel(
        _body,
        out_type=jax.ShapeDtypeStruct((B, T, D), jnp.float32),
        mesh=mesh,
        compiler_params=pltpu.CompilerParams(use_tc_tiling_on_sc=False),
        scratch_types=[
            [pltpu.VMEM((T,), jnp.int32) for _ in range(NBUF)],
            [pltpu.VMEM((T, D), jnp.float32) for _ in range(NBUF)],
            [pltpu.SemaphoreType.DMA for _ in range(NBUF)],
            [pltpu.SemaphoreType.DMA for _ in range(NBUF)],
        ],
    )(table, idx1d)


def kernel(x, table):
    return _emb(table, x.reshape(-1))
